# baseline (speedup 1.0000x reference)
import jax
import jax.numpy as jnp
from jax.experimental import pallas as pl

_BB = 256


def _body(tab_ref, out_ref):
    out_ref[...] = jnp.broadcast_to(tab_ref[...][None, :, :], out_ref.shape)


def kernel(sequence, table):
    b, l = sequence.shape
    d = table.shape[1]
    return pl.pallas_call(
        _body,
        grid=(b // _BB,),
        in_specs=[pl.BlockSpec((l, d), lambda i: (0, 0))],
        out_specs=pl.BlockSpec((_BB, l, d), lambda i: (i, 0, 0)),
        out_shape=jax.ShapeDtypeStruct((b, l, d), table.dtype),
    )(table)
